# Initial kernel scaffold; baseline (speedup 1.0000x reference)
import jax
import jax.numpy as jnp
from jax.experimental import pallas as pl


def kernel(words, chars, edge_index, batch, entity_indices, sent_indices, word_emb, char_emb, conv_w, conv_b, wf_ih, wf_hh, bf_ih, bf_hh, wb_ih, wb_hh, bb_ih, bb_hh, gcn1_w, gcn1_b, gcn2_w, gcn2_b, gcn3_w, gcn3_b, pool1_w, pool1_b, pool2_w, pool2_b, pool3_w, pool3_b):
    raise NotImplementedError("write your pallas kernel here")


# speedup vs baseline: 1.0006x; 1.0006x over previous
import math

import jax
import jax.numpy as jnp
from jax.experimental import pallas as pl

N = 50000
E = 800000
SEQ = 8
WLEN = 10
CFILT = 3
CL = SEQ * (WLEN + CFILT - 1) + CFILT - 1
CFEAT = 32
HID = 32
RATIO = 0.5


def _lstm_dir(x, w_ih, w_hh, b_ih, b_hh):
    nb = x.shape[0]
    h0 = jnp.zeros((nb, HID), x.dtype)
    c0 = jnp.zeros((nb, HID), x.dtype)

    def step(carry, xt):
        h, c = carry
        g = xt @ w_ih.T + b_ih + h @ w_hh.T + b_hh
        i, f, gg, o = jnp.split(g, 4, axis=-1)
        c = jax.nn.sigmoid(f) * c + jax.nn.sigmoid(i) * jnp.tanh(gg)
        h = jax.nn.sigmoid(o) * jnp.tanh(c)
        return (h, c), None

    (hT, cT), _ = jax.lax.scan(step, (h0, c0), jnp.swapaxes(x, 0, 1))
    return hT


def _gcn(x, src, dst, ew, W, b):
    xw = x @ W
    n = x.shape[0]
    deg = jnp.zeros((n,), x.dtype).at[dst].add(ew) + 1.0
    dinv = jax.lax.rsqrt(deg)
    coef = ew * dinv[src] * dinv[dst]
    agg = jnp.zeros_like(xw).at[dst].add(coef[:, None] * xw[src])
    agg = agg + xw * (dinv * dinv)[:, None]
    return agg + b


def _gmp(x):
    return jnp.max(x, axis=0, keepdims=True)


def _gap(x):
    return jnp.mean(x, axis=0, keepdims=True)


def _kgpool(x, src, dst, ew, batch, n1, n2, sidx, w, b):
    n = x.shape[0]
    k = int(math.ceil(RATIO * n))
    score = _gcn(x, src, dst, ew, w, b)[:, 0]
    big = jnp.asarray(1e9, x.dtype)
    score = score.at[n1].set(big).at[n2].set(big).at[sidx].set(big)
    _, perm = jax.lax.top_k(score, k)
    xk = x[perm] * jnp.tanh(score[perm])[:, None]
    mapping = jnp.full((n,), -1, dtype=jnp.int32).at[perm].set(jnp.arange(k, dtype=jnp.int32))
    vs = mapping[src]
    vd = mapping[dst]
    valid = (vs >= 0) & (vd >= 0) & (ew > 0)
    nsrc = jnp.where(valid, vs, 0)
    ndst = jnp.where(valid, vd, 0)
    new_ew = valid.astype(x.dtype)
    return xk, nsrc, ndst, new_ew, batch[perm], mapping[n1], mapping[n2], mapping[sidx]


def _copy_body(x_ref, o_ref):
    o_ref[...] = x_ref[...]


def kernel(words, chars, edge_index, batch, entity_indices, sent_indices,
           word_emb, char_emb, conv_w, conv_b,
           wf_ih, wf_hh, bf_ih, bf_hh, wb_ih, wb_hh, bb_ih, bb_hh,
           gcn1_w, gcn1_b, gcn2_w, gcn2_b, gcn3_w, gcn3_b,
           pool1_w, pool1_b, pool2_w, pool2_b, pool3_w, pool3_b):
    n1 = entity_indices[:, 0]
    n2 = entity_indices[:, 1]
    sidx = jnp.reshape(sent_indices, (-1,))
    we = word_emb[words]
    ce = char_emb[chars]
    ce = jnp.transpose(ce, (0, 2, 1))
    cf = jax.lax.conv_general_dilated(ce, conv_w, (1,), 'VALID',
                                      dimension_numbers=('NCH', 'OIH', 'NCH'))
    cf = cf + conv_b[None, :, None]
    cf = cf.reshape(N, CFEAT, SEQ, WLEN + CFILT - 1).max(axis=-1)
    cf = jnp.tanh(cf)
    cf = jnp.transpose(cf, (0, 2, 1))
    wi = jnp.concatenate([we, cf], axis=-1)
    hf = _lstm_dir(wi, wf_ih, wf_hh, bf_ih, bf_hh)
    hb = _lstm_dir(wi[:, ::-1, :], wb_ih, wb_hh, bb_ih, bb_hh)
    x = jnp.concatenate([hf, hb], axis=-1)
    src = edge_index[0]
    dst = edge_index[1]
    ew = jnp.ones((E,), x.dtype)
    x = jax.nn.relu(_gcn(x, src, dst, ew, gcn1_w, gcn1_b))
    x, src, dst, ew, batch, n1, n2, sidx = _kgpool(x, src, dst, ew, batch, n1, n2, sidx, pool1_w, pool1_b)
    x1 = jnp.concatenate([_gmp(x), _gap(x)], axis=1)
    e1_x1 = x[n1]; e2_x1 = x[n2]; s_x1 = x[sidx]
    x = jax.nn.relu(_gcn(x, src, dst, ew, gcn2_w, gcn2_b))
    x, src, dst, ew, batch, n1, n2, sidx = _kgpool(x, src, dst, ew, batch, n1, n2, sidx, pool2_w, pool2_b)
    x2 = jnp.concatenate([_gmp(x), _gap(x)], axis=1)
    e1_x2 = x[n1]; e2_x2 = x[n2]; s_x2 = x[sidx]
    x = jax.nn.relu(_gcn(x, src, dst, ew, gcn3_w, gcn3_b))
    x, src, dst, ew, batch, n1, n2, sidx = _kgpool(x, src, dst, ew, batch, n1, n2, sidx, pool3_w, pool3_b)
    x3 = jnp.concatenate([_gmp(x), _gap(x)], axis=1)
    e1_x3 = x[n1]; e2_x3 = x[n2]; s_x3 = x[sidx]
    e1_cat = jnp.concatenate([e1_x1, e1_x2, e1_x3], axis=1)
    e2_cat = jnp.concatenate([e2_x1, e2_x2, e2_x3], axis=1)
    s_cat = jnp.concatenate([s_x1, s_x2, s_x3], axis=1)
    xsum = x1 + x2 + x3
    out = jnp.concatenate([e1_cat, e2_cat, s_cat, xsum], axis=1)
    out = pl.pallas_call(
        _copy_body,
        out_shape=jax.ShapeDtypeStruct(out.shape, out.dtype),
    )(out)
    return out


# speedup vs baseline: 1.1808x; 1.1800x over previous
import math

import jax
import jax.numpy as jnp
from jax.experimental import pallas as pl

N = 50000
E = 800000
SEQ = 8
WLEN = 10
CFILT = 3
CL = SEQ * (WLEN + CFILT - 1) + CFILT - 1
CPOS = SEQ * (WLEN + CFILT - 1)
WVOCAB = 30000
WDIM = 128
CVOCAB = 100
CDIM = 32
CFEAT = 32
HID = 32
LSTM_IN = WDIM + CFEAT
RATIO = 0.5


def _nf_body(we_ref, chars_ref, cemb_ref, convw_ref, convb_ref,
             wfih_ref, wfhh_ref, bf_ref, wbih_ref, wbhh_ref, bb_ref,
             out_ref):
    T = we_ref.shape[0]
    R = T * CL
    cemb = cemb_ref[...]
    u = jnp.concatenate(
        [jnp.dot(cemb, convw_ref[:, :, t].T, preferred_element_type=jnp.float32)
         for t in range(CFILT)], axis=1)
    chars_flat = chars_ref[...]
    oh = (chars_flat ==
          jax.lax.broadcasted_iota(jnp.int32, (R, CVOCAB), 1)).astype(jnp.float32)
    a = jnp.dot(oh, u, preferred_element_type=jnp.float32)
    cb = (a[0:R - 2, 0:CFEAT] + a[1:R - 1, CFEAT:2 * CFEAT]
          + a[2:R, 2 * CFEAT:3 * CFEAT])
    cb = jnp.concatenate([cb, jnp.zeros((2, CFEAT), jnp.float32)], axis=0)
    cb = cb.reshape(T, CL, CFEAT)[:, 0:CPOS, :].reshape(T * SEQ, WLEN + CFILT - 1, CFEAT)
    cf = jnp.max(cb, axis=1) + convb_ref[...][None, :]
    cf3 = jnp.tanh(cf).reshape(T, SEQ, CFEAT)

    we3 = we_ref[...]
    masks = [(jax.lax.broadcasted_iota(jnp.int32, (1, SEQ, 1), 1) == t
              ).astype(jnp.float32) for t in range(SEQ)]
    xts = [jnp.concatenate([jnp.sum(we3 * masks[t], axis=1),
                            jnp.sum(cf3 * masks[t], axis=1)], axis=1)
           for t in range(SEQ)]
    wfih = wfih_ref[...]; wfhh = wfhh_ref[...]; bf = bf_ref[...]
    wbih = wbih_ref[...]; wbhh = wbhh_ref[...]; bb = bb_ref[...]

    def run_dir(w_ih, w_hh, b, reverse):
        h = jnp.zeros((T, HID), jnp.float32)
        c = jnp.zeros((T, HID), jnp.float32)
        order = range(SEQ - 1, -1, -1) if reverse else range(SEQ)
        for t in order:
            xt = xts[t]
            g = (jnp.dot(xt, w_ih.T, preferred_element_type=jnp.float32)
                 + jnp.dot(h, w_hh.T, preferred_element_type=jnp.float32) + b[None, :])
            gi = jax.nn.sigmoid(g[:, 0:HID])
            gf = jax.nn.sigmoid(g[:, HID:2 * HID])
            gg = jnp.tanh(g[:, 2 * HID:3 * HID])
            go = jax.nn.sigmoid(g[:, 3 * HID:4 * HID])
            c = gf * c + gi * gg
            h = go * jnp.tanh(c)
        return h

    hf = run_dir(wfih, wfhh, bf, False)
    hb = run_dir(wbih, wbhh, bb, True)
    out_ref[...] = jnp.concatenate([hf, hb], axis=1)


def _node_features(we_flat, chars, char_emb, conv_w, conv_b,
                   wf_ih, wf_hh, bf, wb_ih, wb_hh, bb, tile=80, interpret=False):
    n = we_flat.shape[0]
    we_flat = we_flat.reshape(n, SEQ, WDIM)
    full = lambda shape: pl.BlockSpec(shape, lambda i: tuple(0 for _ in shape))
    return pl.pallas_call(
        _nf_body,
        grid=(n // tile,),
        in_specs=[
            pl.BlockSpec((tile, SEQ, WDIM), lambda i: (i, 0, 0)),
            pl.BlockSpec((tile * CL, 1), lambda i: (i, 0)),
            full(char_emb.shape), full(conv_w.shape), full(conv_b.shape),
            full(wf_ih.shape), full(wf_hh.shape), full(bf.shape),
            full(wb_ih.shape), full(wb_hh.shape), full(bb.shape),
        ],
        out_specs=pl.BlockSpec((tile, 2 * HID), lambda i: (i, 0)),
        out_shape=jax.ShapeDtypeStruct((n, 2 * HID), jnp.float32),
        interpret=interpret,
    )(we_flat, chars.reshape(n * CL, 1), char_emb, conv_w, conv_b,
      wf_ih, wf_hh, bf, wb_ih, wb_hh, bb)


def _gcn(x, src, dst, ew, W, b):
    xw = x @ W
    n = x.shape[0]
    deg = jnp.zeros((n,), x.dtype).at[dst].add(ew) + 1.0
    dinv = jax.lax.rsqrt(deg)
    coef = ew * dinv[src] * dinv[dst]
    agg = jnp.zeros_like(xw).at[dst].add(coef[:, None] * xw[src])
    agg = agg + xw * (dinv * dinv)[:, None]
    return agg + b


def _kgpool(x, src, dst, ew, n1, n2, sidx, w, b):
    n = x.shape[0]
    k = int(math.ceil(RATIO * n))
    score = _gcn(x, src, dst, ew, w, b)[:, 0]
    big = jnp.asarray(1e9, x.dtype)
    score = score.at[n1].set(big).at[n2].set(big).at[sidx].set(big)
    _, perm = jax.lax.top_k(score, k)
    xk = x[perm] * jnp.tanh(score[perm])[:, None]
    mapping = jnp.full((n,), -1, dtype=jnp.int32).at[perm].set(jnp.arange(k, dtype=jnp.int32))
    vs = mapping[src]
    vd = mapping[dst]
    valid = (vs >= 0) & (vd >= 0) & (ew > 0)
    nsrc = jnp.where(valid, vs, 0)
    ndst = jnp.where(valid, vd, 0)
    new_ew = valid.astype(x.dtype)
    return xk, nsrc, ndst, new_ew, mapping[n1], mapping[n2], mapping[sidx]


def kernel(words, chars, edge_index, batch, entity_indices, sent_indices,
           word_emb, char_emb, conv_w, conv_b,
           wf_ih, wf_hh, bf_ih, bf_hh, wb_ih, wb_hh, bb_ih, bb_hh,
           gcn1_w, gcn1_b, gcn2_w, gcn2_b, gcn3_w, gcn3_b,
           pool1_w, pool1_b, pool2_w, pool2_b, pool3_w, pool3_b):
    n1 = entity_indices[:, 0]
    n2 = entity_indices[:, 1]
    sidx = jnp.reshape(sent_indices, (-1,))

    we_flat = word_emb[words].reshape(N, SEQ * WDIM)
    x = _node_features(we_flat, chars, char_emb, conv_w, conv_b,
                       wf_ih, wf_hh, bf_ih + bf_hh,
                       wb_ih, wb_hh, bb_ih + bb_hh)

    src = edge_index[0]
    dst = edge_index[1]
    ew = jnp.ones((E,), x.dtype)
    outs = []
    for gw, gb, pw, pb in ((gcn1_w, gcn1_b, pool1_w, pool1_b),
                           (gcn2_w, gcn2_b, pool2_w, pool2_b),
                           (gcn3_w, gcn3_b, pool3_w, pool3_b)):
        x = jax.nn.relu(_gcn(x, src, dst, ew, gw, gb))
        x, src, dst, ew, n1, n2, sidx = _kgpool(x, src, dst, ew, n1, n2, sidx, pw, pb)
        xb = jnp.concatenate([jnp.max(x, axis=0, keepdims=True),
                              jnp.mean(x, axis=0, keepdims=True)], axis=1)
        outs.append((xb, x[n1], x[n2], x[sidx]))
    e1_cat = jnp.concatenate([o[1] for o in outs], axis=1)
    e2_cat = jnp.concatenate([o[2] for o in outs], axis=1)
    s_cat = jnp.concatenate([o[3] for o in outs], axis=1)
    xsum = outs[0][0] + outs[1][0] + outs[2][0]
    return jnp.concatenate([e1_cat, e2_cat, s_cat, xsum], axis=1)
